# baseline (speedup 1.0000x reference)
import functools

import jax
import jax.numpy as jnp
from jax import lax
from jax.experimental import pallas as pl
from jax.experimental.pallas import tpu as pltpu
from jax.experimental.pallas import tpu_sc as plsc

D_MODEL = 64
_SCALE = 8.0

_NC = 2
_NS = 16
_NW = _NC * _NS
_L = 16

_C = 128
_K = 4


@functools.lru_cache(maxsize=None)
def _build(B: int):
    bpw = B // _NW
    nch = bpw // _C
    nblk = nch // _K
    assert bpw * _NW == B and nch * _C == bpw and nblk * _K == nch

    mesh = plsc.VectorSubcoreMesh(
        core_axis_name="c", subcore_axis_name="s",
        num_cores=_NC, num_subcores=_NS,
    )

    @functools.partial(
        pl.kernel,
        mesh=mesh,
        out_type=jax.ShapeDtypeStruct((B, D_MODEL), jnp.float32),
        scratch_types=[
            pltpu.VMEM((nch, _C), jnp.int32),
            [pltpu.VMEM((_C, D_MODEL), jnp.float32) for _ in range(_K)],
            [pltpu.SemaphoreType.DMA for _ in range(_K)],
            [pltpu.SemaphoreType.DMA for _ in range(_K)],
        ],
        compiler_params=pltpu.CompilerParams(use_tc_tiling_on_sc=False),
    )
    def kern(x_hbm, table_hbm, out_hbm, idx_v, bufs, gsems, osems):
        cid = lax.axis_index("c")
        sid = lax.axis_index("s")
        wid = sid * _NC + cid
        base = wid * bpw

        pltpu.sync_copy(x_hbm.at[wid], idx_v)

        def start_gather(g, b):
            pltpu.async_copy(table_hbm.at[idx_v.at[g]], bufs[b], gsems[b])

        def wait_gather(b):
            pltpu.make_async_copy(
                table_hbm.at[idx_v.at[0]], bufs[b], gsems[b]
            ).wait()

        def start_out(g, b):
            pltpu.async_copy(
                bufs[b], out_hbm.at[pl.ds(base + g * _C, _C)], osems[b]
            )

        def wait_out(b):
            pltpu.make_async_copy(
                bufs[b], out_hbm.at[pl.ds(base, _C)], osems[b]
            ).wait()

        def scale(b):
            buf = bufs[b]

            @pl.loop(0, _C)
            def _(r):
                for j in range(D_MODEL // _L):
                    sl = pl.ds(j * _L, _L)
                    buf[r, sl] = buf[r, sl] * _SCALE

        def step(g, b, *, first=False, prefetch=True):
            if not first:
                wait_out((b + _K - 1) % _K)
            if prefetch:
                start_gather(g + _K - 1, (b + _K - 1) % _K)
            wait_gather(b)
            scale(b)
            start_out(g, b)

        for b in range(_K - 1):
            start_gather(b, b)

        step(0, 0, first=True)
        for b in range(1, _K):
            step(b, b)

        @pl.loop(1, nblk - 1)
        def _(i):
            g0 = i * _K
            for b in range(_K):
                step(g0 + b, b)

        g0 = (nblk - 1) * _K
        step(g0, 0)
        for b in range(1, _K):
            step(g0 + b, b, prefetch=False)

        wait_out(_K - 1)

    return kern


def kernel(x, table):
    s0, s1 = x.shape
    B = s0 * s1
    xw = x.reshape(_NW, B // _NW // _C, _C).astype(jnp.int32)
    out = _build(B)(xw, table)
    return out.reshape(s0, s1, D_MODEL)
